# Initial kernel scaffold; baseline (speedup 1.0000x reference)
import jax
import jax.numpy as jnp
from jax.experimental import pallas as pl


def kernel(hidden_states, gate_weight):
    raise NotImplementedError("write your pallas kernel here")


# speedup vs baseline: 1.1414x; 1.1414x over previous
import functools

import jax
import jax.numpy as jnp
from jax import lax
from jax.experimental import pallas as pl
from jax.experimental.pallas import tpu as pltpu

NUM_EXPERTS = 64
TOP_K = 8
HIDDEN = 2048
TOKENS = 16384

TOKEN_BLOCK = 1024


def _router_body(h_ref, w_ref, logits_ref, weights_ref, experts_ref):
    logits = lax.dot_general(
        h_ref[...], w_ref[...],
        dimension_numbers=(((1,), (1,)), ((), ())),
        preferred_element_type=jnp.float32,
    )
    logits_ref[...] = logits

    iota_e = lax.broadcasted_iota(jnp.int32, logits.shape, 1)
    work = logits
    topv = []
    topi = []
    for _ in range(TOP_K):
        m = jnp.max(work, axis=-1, keepdims=True)
        idx = jnp.min(jnp.where(work == m, iota_e, NUM_EXPERTS), axis=-1,
                      keepdims=True)
        topv.append(m)
        topi.append(idx)
        work = jnp.where(iota_e == idx, -jnp.inf, work)
    vals = jnp.concatenate(topv, axis=-1)
    idxs = jnp.concatenate(topi, axis=-1)
    e = jnp.exp(vals - vals[:, :1])
    weights_ref[...] = e / jnp.sum(e, axis=-1, keepdims=True)
    experts_ref[...] = idxs


@jax.jit
def kernel(hidden_states, gate_weight):
    n_blocks = TOKENS // TOKEN_BLOCK
    logits, weights, experts = pl.pallas_call(
        _router_body,
        grid=(n_blocks,),
        in_specs=[
            pl.BlockSpec((TOKEN_BLOCK, HIDDEN), lambda i: (i, 0)),
            pl.BlockSpec((NUM_EXPERTS, HIDDEN), lambda i: (0, 0)),
        ],
        out_specs=[
            pl.BlockSpec((TOKEN_BLOCK, NUM_EXPERTS), lambda i: (i, 0)),
            pl.BlockSpec((TOKEN_BLOCK, TOP_K), lambda i: (i, 0)),
            pl.BlockSpec((TOKEN_BLOCK, TOP_K), lambda i: (i, 0)),
        ],
        out_shape=[
            jax.ShapeDtypeStruct((TOKENS, NUM_EXPERTS), jnp.float32),
            jax.ShapeDtypeStruct((TOKENS, TOP_K), jnp.float32),
            jax.ShapeDtypeStruct((TOKENS, TOP_K), jnp.int32),
        ],
        compiler_params=pltpu.CompilerParams(
            dimension_semantics=("arbitrary",),
        ),
    )(hidden_states, gate_weight)
    return weights, experts, logits
